# baseline (speedup 1.0000x reference)
import functools

import jax
import jax.numpy as jnp
from jax import lax
from jax.experimental import pallas as pl
from jax.experimental.pallas import tpu as pltpu
from jax.experimental.pallas import tpu_sc as plsc

_N = 10000
_D = 256
_H = 4
_DK = 64
_I = 5

_NW = 32
_PER_W = 320
_NPAD = _NW * _PER_W
_CHUNK = 80
_NCHUNK = _PER_W // _CHUNK

_NT_DIMS = (((1,), (1,)), ((), ()))


@functools.lru_cache(maxsize=None)
def _make_sc_gather():
    @functools.partial(
        pl.kernel,
        mesh=plsc.VectorSubcoreMesh(core_axis_name="c", subcore_axis_name="s"),
        out_type=jax.ShapeDtypeStruct((_N, _D // 2), jnp.int32),
        scratch_types=[
            pltpu.VMEM((_PER_W,), jnp.int32),
            pltpu.VMEM((_CHUNK, _D // 2), jnp.int32),
            pltpu.VMEM((_CHUNK, _D // 2), jnp.int32),
            pltpu.SemaphoreType.DMA,
            pltpu.SemaphoreType.DMA,
            pltpu.SemaphoreType.DMA,
            pltpu.SemaphoreType.DMA,
        ],
    )
    def _sc_gather(table_hbm, idx_hbm, out_hbm, idx_v,
                   rows_a, rows_b, gs_a, gs_b, ws_a, ws_b):
        wid = lax.axis_index("s") * 2 + lax.axis_index("c")
        base = wid * _PER_W
        pltpu.sync_copy(idx_hbm.at[pl.ds(base, _PER_W)], idx_v)

        rows = (rows_a, rows_b)
        gs = (gs_a, gs_b)
        ws = (ws_a, ws_b)
        offs = [pl.multiple_of(base + c * _CHUNK, 16) for c in range(_NCHUNK)]
        conds = [offs[c] < _N for c in range(_NCHUNK)]

        def gstart(c):
            b = c % 2
            pltpu.async_copy(
                table_hbm.at[idx_v.at[pl.ds(c * _CHUNK, _CHUNK)]],
                rows[b], gs[b])

        def gwait(c):
            b = c % 2
            pltpu.make_async_copy(
                table_hbm.at[idx_v.at[pl.ds(c * _CHUNK, _CHUNK)]],
                rows[b], gs[b]).wait()

        def wstart(c):
            b = c % 2
            pltpu.async_copy(rows[b], out_hbm.at[pl.ds(offs[c], _CHUNK)],
                             ws[b])

        def wwait(c):
            b = c % 2
            pltpu.make_async_copy(rows[b], out_hbm.at[pl.ds(offs[c], _CHUNK)],
                                  ws[b]).wait()

        for c in range(_NCHUNK):
            if c >= 2:
                @pl.when(conds[c - 2])
                def _(c=c):
                    wwait(c - 2)

            @pl.when(conds[c])
            def _(c=c):
                gstart(c)

            @pl.when(conds[c])
            def _(c=c):
                gwait(c)
                wstart(c)
        for c in range(max(_NCHUNK - 2, 0), _NCHUNK):
            @pl.when(conds[c])
            def _(c=c):
                wwait(c)

    return _sc_gather


def _q_body(x_ref, wqt_ref, q_ref):
    q_ref[0] = lax.dot_general(
        wqt_ref[0].astype(jnp.bfloat16), x_ref[...].astype(jnp.bfloat16),
        _NT_DIMS, preferred_element_type=jnp.float32)


def _tc_q(x, wqt):
    return pl.pallas_call(
        _q_body,
        grid=(_H,),
        in_specs=[
            pl.BlockSpec((_N, _D), lambda h: (0, 0)),
            pl.BlockSpec((1, _DK, _D), lambda h: (h, 0, 0)),
        ],
        out_specs=pl.BlockSpec((1, _DK, _N), lambda h: (h, 0, 0)),
        out_shape=jax.ShapeDtypeStruct((_H, _DK, _N), jnp.float32),
    )(x, wqt)


def _unpack_to_scratch(xn_ref, xbf_ref):
    @pl.when(pl.program_id(0) == 0)
    def _():
        xi = xn_ref[...]
        lo = lax.shift_left(xi, 16)
        hi = lax.bitwise_and(xi, jnp.int32(-65536))
        xbf_ref[:, :_D // 2] = lax.bitcast_convert_type(
            lo, jnp.float32).astype(jnp.bfloat16)
        xbf_ref[:, _D // 2:] = lax.bitcast_convert_type(
            hi, jnp.float32).astype(jnp.bfloat16)


def _k_body_first(xn_ref, wkt_ref, k_ref, xbf_ref):
    _unpack_to_scratch(xn_ref, xbf_ref)
    k_ref[0, 0] = lax.dot_general(
        wkt_ref[0, 0].astype(jnp.bfloat16), xbf_ref[...],
        _NT_DIMS, preferred_element_type=jnp.float32)


def _k_body_acc(kin_ref, xn_ref, wkt_ref, k_ref, xbf_ref):
    del kin_ref
    _unpack_to_scratch(xn_ref, xbf_ref)
    k_ref[0, 0] = lax.dot_general(
        wkt_ref[0, 0].astype(jnp.bfloat16), xbf_ref[...],
        _NT_DIMS, preferred_element_type=jnp.float32)


def _tc_k_slot(i, kt, xn_i, wkt):
    out_spec = pl.BlockSpec((1, 1, _DK, _N), lambda h: (h, i, 0, 0))
    out_shape = jax.ShapeDtypeStruct((_H, _I, _DK, _N), jnp.float32)
    common = [
        pl.BlockSpec((_N, _D // 2), lambda h: (0, 0)),
        pl.BlockSpec((1, 1, _DK, _D), lambda h, _i=i: (h, _i, 0, 0)),
    ]
    scratch = [pltpu.VMEM((_N, _D), jnp.bfloat16)]
    if kt is None:
        return pl.pallas_call(
            _k_body_first,
            grid=(_H,),
            in_specs=common,
            out_specs=out_spec,
            out_shape=out_shape,
            scratch_shapes=scratch,
        )(xn_i, wkt)
    return pl.pallas_call(
        _k_body_acc,
        grid=(_H,),
        in_specs=[pl.BlockSpec(memory_space=pl.ANY)] + common,
        out_specs=out_spec,
        out_shape=out_shape,
        input_output_aliases={0: 0},
        scratch_shapes=scratch,
    )(kt, xn_i, wkt)


def kernel(x, adjacency_matrix, w_q, w_k, w_v):
    del w_v
    idx = jnp.pad(adjacency_matrix, ((0, 0), (0, _NPAD - _N)))
    x_packed = lax.bitcast_convert_type(
        x.astype(jnp.bfloat16).reshape(_N, _D // 2, 2), jnp.int32)
    gather = _make_sc_gather()
    xns = [gather(x_packed, idx[i]) for i in range(_I)]
    qt = _tc_q(x, w_q.transpose(0, 2, 1))
    wkt = w_k.transpose(0, 1, 3, 2)
    wkt = jnp.concatenate([wkt[..., 0::2], wkt[..., 1::2]], axis=-1)
    kt = None
    for i in range(_I):
        kt = _tc_k_slot(i, kt, xns[i], wkt)
    return (qt.transpose(0, 2, 1), kt.transpose(0, 1, 3, 2))


# speedup vs baseline: 2.3327x; 2.3327x over previous
import functools

import jax
import jax.numpy as jnp
from jax import lax
from jax.experimental import pallas as pl
from jax.experimental.pallas import tpu as pltpu
from jax.experimental.pallas import tpu_sc as plsc

_N = 10000
_D = 256
_H = 4
_DK = 64
_I = 5

_NW = 32
_PER_W = 320
_NPAD = _NW * _PER_W
_CHUNK = 80
_NCHUNK = _PER_W // _CHUNK

_NT_DIMS = (((1,), (1,)), ((), ()))


@functools.lru_cache(maxsize=None)
def _make_sc_gather():
    @functools.partial(
        pl.kernel,
        mesh=plsc.VectorSubcoreMesh(core_axis_name="c", subcore_axis_name="s"),
        out_type=jax.ShapeDtypeStruct((_N, _D // 2), jnp.int32),
        scratch_types=[
            pltpu.VMEM((_PER_W,), jnp.int32),
            pltpu.VMEM((_CHUNK, _D // 2), jnp.int32),
            pltpu.VMEM((_CHUNK, _D // 2), jnp.int32),
            pltpu.SemaphoreType.DMA,
            pltpu.SemaphoreType.DMA,
            pltpu.SemaphoreType.DMA,
            pltpu.SemaphoreType.DMA,
        ],
    )
    def _sc_gather(table_hbm, idx_hbm, out_hbm, idx_v,
                   rows_a, rows_b, gs_a, gs_b, ws_a, ws_b):
        wid = lax.axis_index("s") * 2 + lax.axis_index("c")
        base = wid * _PER_W
        pltpu.sync_copy(idx_hbm.at[pl.ds(base, _PER_W)], idx_v)

        rows = (rows_a, rows_b)
        gs = (gs_a, gs_b)
        ws = (ws_a, ws_b)
        offs = [pl.multiple_of(base + c * _CHUNK, 16) for c in range(_NCHUNK)]
        conds = [offs[c] < _N for c in range(_NCHUNK)]

        def gstart(c):
            b = c % 2
            pltpu.async_copy(
                table_hbm.at[idx_v.at[pl.ds(c * _CHUNK, _CHUNK)]],
                rows[b], gs[b])

        def gwait(c):
            b = c % 2
            pltpu.make_async_copy(
                table_hbm.at[idx_v.at[pl.ds(c * _CHUNK, _CHUNK)]],
                rows[b], gs[b]).wait()

        def wstart(c):
            b = c % 2
            pltpu.async_copy(rows[b], out_hbm.at[pl.ds(offs[c], _CHUNK)],
                             ws[b])

        def wwait(c):
            b = c % 2
            pltpu.make_async_copy(rows[b], out_hbm.at[pl.ds(offs[c], _CHUNK)],
                                  ws[b]).wait()

        for c in range(_NCHUNK):
            if c >= 2:
                @pl.when(conds[c - 2])
                def _(c=c):
                    wwait(c - 2)

            @pl.when(conds[c])
            def _(c=c):
                gstart(c)

            @pl.when(conds[c])
            def _(c=c):
                gwait(c)
                wstart(c)
        for c in range(max(_NCHUNK - 2, 0), _NCHUNK):
            @pl.when(conds[c])
            def _(c=c):
                wwait(c)

    return _sc_gather


def _q_body(x_ref, wqt_ref, q_ref):
    q_ref[0] = lax.dot_general(
        wqt_ref[0].astype(jnp.bfloat16), x_ref[...].astype(jnp.bfloat16),
        _NT_DIMS, preferred_element_type=jnp.float32)


def _tc_q(x, wqt):
    return pl.pallas_call(
        _q_body,
        grid=(_H,),
        in_specs=[
            pl.BlockSpec((_N, _D), lambda h: (0, 0)),
            pl.BlockSpec((1, _DK, _D), lambda h: (h, 0, 0)),
        ],
        out_specs=pl.BlockSpec((1, _DK, _N), lambda h: (h, 0, 0)),
        out_shape=jax.ShapeDtypeStruct((_H, _DK, _N), jnp.float32),
    )(x, wqt)


def _rne_bf16_bits(u):
    sixteen = jnp.uint32(16)
    odd = lax.bitwise_and(lax.shift_right_logical(u, sixteen), jnp.uint32(1))
    return lax.shift_right_logical(u + jnp.uint32(0x7FFF) + odd, sixteen)


def _pack_body(x_ref, out_ref):
    u = lax.bitcast_convert_type(x_ref[...], jnp.uint32)
    b = _rne_bf16_bits(u)
    w = lax.bitwise_or(b[:, :_D // 2],
                       lax.shift_left(b[:, _D // 2:], jnp.uint32(16)))
    out_ref[...] = lax.bitcast_convert_type(w, jnp.int32)


def _tc_pack(x):
    bn = 2000
    return pl.pallas_call(
        _pack_body,
        grid=(_N // bn,),
        in_specs=[pl.BlockSpec((bn, _D), lambda n: (n, 0))],
        out_specs=pl.BlockSpec((bn, _D // 2), lambda n: (n, 0)),
        out_shape=jax.ShapeDtypeStruct((_N, _D // 2), jnp.int32),
    )(x)


def _unpack_to_scratch(xn_ref, xbf_ref):
    @pl.when(pl.program_id(0) == 0)
    def _():
        xi = xn_ref[...]
        lo = lax.shift_left(xi, 16)
        hi = lax.bitwise_and(xi, jnp.int32(-65536))
        xbf_ref[:, :_D // 2] = lax.bitcast_convert_type(
            lo, jnp.float32).astype(jnp.bfloat16)
        xbf_ref[:, _D // 2:] = lax.bitcast_convert_type(
            hi, jnp.float32).astype(jnp.bfloat16)


def _k_body_first(xn_ref, wkt_ref, k_ref, xbf_ref):
    _unpack_to_scratch(xn_ref, xbf_ref)
    k_ref[0, 0] = lax.dot_general(
        wkt_ref[0, 0].astype(jnp.bfloat16), xbf_ref[...],
        _NT_DIMS, preferred_element_type=jnp.float32)


def _k_body_acc(kin_ref, xn_ref, wkt_ref, k_ref, xbf_ref):
    del kin_ref
    _unpack_to_scratch(xn_ref, xbf_ref)
    k_ref[0, 0] = lax.dot_general(
        wkt_ref[0, 0].astype(jnp.bfloat16), xbf_ref[...],
        _NT_DIMS, preferred_element_type=jnp.float32)


def _tc_k_slot(i, kt, xn_i, wkt):
    out_spec = pl.BlockSpec((1, 1, _DK, _N), lambda h: (h, i, 0, 0))
    out_shape = jax.ShapeDtypeStruct((_H, _I, _DK, _N), jnp.float32)
    common = [
        pl.BlockSpec((_N, _D // 2), lambda h: (0, 0)),
        pl.BlockSpec((1, 1, _DK, _D), lambda h, _i=i: (h, _i, 0, 0)),
    ]
    scratch = [pltpu.VMEM((_N, _D), jnp.bfloat16)]
    if kt is None:
        return pl.pallas_call(
            _k_body_first,
            grid=(_H,),
            in_specs=common,
            out_specs=out_spec,
            out_shape=out_shape,
            scratch_shapes=scratch,
        )(xn_i, wkt)
    return pl.pallas_call(
        _k_body_acc,
        grid=(_H,),
        in_specs=[pl.BlockSpec(memory_space=pl.ANY)] + common,
        out_specs=out_spec,
        out_shape=out_shape,
        input_output_aliases={0: 0},
        scratch_shapes=scratch,
    )(kt, xn_i, wkt)


def kernel(x, adjacency_matrix, w_q, w_k, w_v):
    del w_v
    idx = jnp.pad(adjacency_matrix, ((0, 0), (0, _NPAD - _N)))
    x_packed = _tc_pack(x)
    gather = _make_sc_gather()
    xns = [gather(x_packed, idx[i]) for i in range(_I)]
    qt = _tc_q(x, w_q.transpose(0, 2, 1))
    wkt = w_k.transpose(0, 1, 3, 2)
    kt = None
    for i in range(_I):
        kt = _tc_k_slot(i, kt, xns[i], wkt)
    return (qt.transpose(0, 2, 1), kt.transpose(0, 1, 3, 2))
